# baseline (device time: 52438 ns/iter reference)
import jax
import jax.numpy as jnp
from jax import lax
from jax.experimental import pallas as pl
from jax.experimental.pallas import tpu as pltpu

N_DEV = 32
B, SQ, SKV, D_MODEL = 2, 256, 512, 768
H_LOC, DH = 8, 64
ROWS = B * SQ
CHUNK = ROWS // N_DEV


def _body(
    x_ref, wq_ref, wo_ref, k_ref, v_ref, out_ref,
    o_ref, part_ref, conv_ref, rs_buf, ag_buf,
    rs_send, rs_recv, ag_send, ag_recv,
):
    my = lax.axis_index("i")

    qx = x_ref[...].astype(jnp.bfloat16)
    wq = wq_ref[...].astype(jnp.bfloat16)
    q = jnp.dot(qx, wq, preferred_element_type=jnp.float32)
    q = (q * 0.125).astype(jnp.bfloat16)

    for b in range(B):
        for h in range(H_LOC):
            qbh = q[b * SQ:(b + 1) * SQ, h * DH:(h + 1) * DH]
            kbh = k_ref[b, :, h, :].astype(jnp.bfloat16)
            s = lax.dot_general(
                qbh, kbh, (((1,), (1,)), ((), ())),
                preferred_element_type=jnp.float32,
            )
            m = jnp.max(s, axis=-1, keepdims=True)
            e = jnp.exp(s - m)
            l = jnp.sum(e, axis=-1, keepdims=True)
            p = (e / l).astype(jnp.bfloat16)
            vbh = v_ref[b, :, h, :].astype(jnp.bfloat16)
            obh = jnp.dot(p, vbh, preferred_element_type=jnp.float32)
            o_ref[b * SQ:(b + 1) * SQ, h * DH:(h + 1) * DH] = (
                obh.astype(jnp.bfloat16)
            )

    wo = wo_ref[...].astype(jnp.bfloat16)
    partial = jnp.dot(o_ref[...], wo, preferred_element_type=jnp.float32)
    part_ref[...] = partial.reshape(N_DEV, CHUNK, D_MODEL)
    conv_ref[...] = part_ref[...].astype(jnp.bfloat16)

    barrier = pltpu.get_barrier_semaphore()
    for d in range(1, N_DEV):
        pl.semaphore_signal(
            barrier, inc=1,
            device_id=(lax.rem(my + d, N_DEV),),
            device_id_type=pl.DeviceIdType.MESH,
        )
    pl.semaphore_wait(barrier, N_DEV - 1)

    rs = []
    for d in range(1, N_DEV):
        tgt = lax.rem(my + d, N_DEV)
        rdma = pltpu.make_async_remote_copy(
            src_ref=conv_ref.at[tgt],
            dst_ref=rs_buf.at[N_DEV - d],
            send_sem=rs_send.at[d],
            recv_sem=rs_recv.at[N_DEV - d],
            device_id=(tgt,),
            device_id_type=pl.DeviceIdType.MESH,
        )
        rdma.start()
        rs.append(rdma)
    for rdma in rs:
        rdma.wait_recv()
    for rdma in rs:
        rdma.wait_send()

    acc = part_ref[my]
    for j in range(1, N_DEV):
        acc = acc + rs_buf[j].astype(jnp.float32)
    ag_buf[my] = acc.astype(jnp.bfloat16)

    ag = []
    for d in range(1, N_DEV):
        tgt = lax.rem(my + d, N_DEV)
        rdma = pltpu.make_async_remote_copy(
            src_ref=ag_buf.at[my],
            dst_ref=ag_buf.at[my],
            send_sem=ag_send.at[d],
            recv_sem=ag_recv.at[N_DEV - d],
            device_id=(tgt,),
            device_id_type=pl.DeviceIdType.MESH,
        )
        rdma.start()
        ag.append(rdma)
    for rdma in ag:
        rdma.wait_recv()
    for rdma in ag:
        rdma.wait_send()

    out_ref[...] = ag_buf[...].astype(jnp.float32)


def kernel(x, Wq, Wo, K_ext, V_ext):
    out = pl.pallas_call(
        _body,
        out_shape=jax.ShapeDtypeStruct((N_DEV, CHUNK, D_MODEL), jnp.float32),
        in_specs=[pl.BlockSpec(memory_space=pltpu.VMEM)] * 5,
        out_specs=pl.BlockSpec(memory_space=pltpu.VMEM),
        scratch_shapes=[
            pltpu.VMEM((ROWS, H_LOC * DH), jnp.bfloat16),
            pltpu.VMEM((N_DEV, CHUNK, D_MODEL), jnp.float32),
            pltpu.VMEM((N_DEV, CHUNK, D_MODEL), jnp.bfloat16),
            pltpu.VMEM((N_DEV, CHUNK, D_MODEL), jnp.bfloat16),
            pltpu.VMEM((N_DEV, CHUNK, D_MODEL), jnp.bfloat16),
            pltpu.SemaphoreType.DMA((N_DEV,)),
            pltpu.SemaphoreType.DMA((N_DEV,)),
            pltpu.SemaphoreType.DMA((N_DEV,)),
            pltpu.SemaphoreType.DMA((N_DEV,)),
        ],
        compiler_params=pltpu.CompilerParams(collective_id=0),
    )(x.reshape(ROWS, D_MODEL), Wq, Wo, K_ext, V_ext)
    return out.reshape(B, SQ, D_MODEL)


# device time: 44412 ns/iter; 1.1807x vs baseline; 1.1807x over previous
import jax
import jax.numpy as jnp
from jax import lax
from jax.experimental import pallas as pl
from jax.experimental.pallas import tpu as pltpu

N_DEV = 32
B, SQ, SKV, D_MODEL = 2, 256, 512, 768
H_LOC, DH = 8, 64
ROWS = B * SQ
CHUNK = ROWS // N_DEV


def _body(
    x_ref, wq_ref, wo_ref, k_ref, v_ref, out_ref,
    o_ref, part_ref, conv_ref, rs_buf, ag_buf,
    rs_send, rs_recv, ag_send, ag_recv,
):
    my = lax.axis_index("i")

    qx = x_ref[...].astype(jnp.bfloat16)
    wq = wq_ref[...].astype(jnp.bfloat16)
    q = jnp.dot(qx, wq, preferred_element_type=jnp.float32)
    q = (q * 0.125).astype(jnp.bfloat16)

    for b in range(B):
        for h in range(H_LOC):
            qbh = q[b * SQ:(b + 1) * SQ, h * DH:(h + 1) * DH]
            kbh = k_ref[b, h].astype(jnp.bfloat16)
            s = lax.dot_general(
                qbh, kbh, (((1,), (1,)), ((), ())),
                preferred_element_type=jnp.float32,
            )
            m = jnp.max(s, axis=-1, keepdims=True)
            e = jnp.exp(s - m)
            l = jnp.sum(e, axis=-1, keepdims=True)
            p = e.astype(jnp.bfloat16)
            vbh = v_ref[b, h].astype(jnp.bfloat16)
            obh = jnp.dot(p, vbh, preferred_element_type=jnp.float32)
            o_ref[b * SQ:(b + 1) * SQ, h * DH:(h + 1) * DH] = (
                (obh / l).astype(jnp.bfloat16)
            )

    wo = wo_ref[...].astype(jnp.bfloat16)
    partial = jnp.dot(o_ref[...], wo, preferred_element_type=jnp.float32)
    part_ref[...] = partial.reshape(N_DEV, CHUNK, D_MODEL)
    conv_ref[...] = part_ref[...].astype(jnp.bfloat16)

    barrier = pltpu.get_barrier_semaphore()
    for d in range(1, N_DEV):
        pl.semaphore_signal(
            barrier, inc=1,
            device_id=(lax.rem(my + d, N_DEV),),
            device_id_type=pl.DeviceIdType.MESH,
        )
    pl.semaphore_wait(barrier, N_DEV - 1)

    rs = []
    for d in range(1, N_DEV):
        tgt = lax.rem(my + d, N_DEV)
        rdma = pltpu.make_async_remote_copy(
            src_ref=conv_ref.at[tgt],
            dst_ref=rs_buf.at[N_DEV - d],
            send_sem=rs_send.at[d],
            recv_sem=rs_recv.at[N_DEV - d],
            device_id=(tgt,),
            device_id_type=pl.DeviceIdType.MESH,
        )
        rdma.start()
        rs.append(rdma)
    for rdma in rs:
        rdma.wait_recv()
    for rdma in rs:
        rdma.wait_send()

    acc = part_ref[my]
    for j in range(1, N_DEV):
        acc = acc + rs_buf[j].astype(jnp.float32)
    ag_buf[my] = acc.astype(jnp.bfloat16)

    ag = []
    for d in range(1, N_DEV):
        tgt = lax.rem(my + d, N_DEV)
        rdma = pltpu.make_async_remote_copy(
            src_ref=ag_buf.at[my],
            dst_ref=ag_buf.at[my],
            send_sem=ag_send.at[d],
            recv_sem=ag_recv.at[N_DEV - d],
            device_id=(tgt,),
            device_id_type=pl.DeviceIdType.MESH,
        )
        rdma.start()
        ag.append(rdma)
    for rdma in ag:
        rdma.wait_recv()
    for rdma in ag:
        rdma.wait_send()

    out_ref[...] = ag_buf[...].astype(jnp.float32)


def kernel(x, Wq, Wo, K_ext, V_ext):
    out = pl.pallas_call(
        _body,
        out_shape=jax.ShapeDtypeStruct((N_DEV, CHUNK, D_MODEL), jnp.float32),
        in_specs=[pl.BlockSpec(memory_space=pltpu.VMEM)] * 5,
        out_specs=pl.BlockSpec(memory_space=pltpu.VMEM),
        scratch_shapes=[
            pltpu.VMEM((ROWS, H_LOC * DH), jnp.bfloat16),
            pltpu.VMEM((N_DEV, CHUNK, D_MODEL), jnp.float32),
            pltpu.VMEM((N_DEV, CHUNK, D_MODEL), jnp.bfloat16),
            pltpu.VMEM((N_DEV, CHUNK, D_MODEL), jnp.bfloat16),
            pltpu.VMEM((N_DEV, CHUNK, D_MODEL), jnp.bfloat16),
            pltpu.SemaphoreType.DMA((N_DEV,)),
            pltpu.SemaphoreType.DMA((N_DEV,)),
            pltpu.SemaphoreType.DMA((N_DEV,)),
            pltpu.SemaphoreType.DMA((N_DEV,)),
        ],
        compiler_params=pltpu.CompilerParams(collective_id=0),
    )(
        x.reshape(ROWS, D_MODEL),
        Wq,
        Wo,
        K_ext.transpose(0, 2, 1, 3),
        V_ext.transpose(0, 2, 1, 3),
    )
    return out.reshape(B, SQ, D_MODEL)
